# Initial kernel scaffold; baseline (speedup 1.0000x reference)
import jax
import jax.numpy as jnp
from jax.experimental import pallas as pl


def kernel(input_ids, table):
    raise NotImplementedError("write your pallas kernel here")


# speedup vs baseline: 1.7960x; 1.7960x over previous
import functools

import jax
import jax.numpy as jnp
from jax import lax
from jax.experimental import pallas as pl
from jax.experimental.pallas import tpu as pltpu
from jax.experimental.pallas import tpu_sc as plsc

_VOCAB = 1000000
_EMBED_DIM = 64
_BATCH = 16384
_HIST = 50
_B = _BATCH * _HIST

_info = plsc.get_sparse_core_info()
_NC = _info.num_cores
_NS = _info.num_subcores
_NW = _NC * _NS
_BPW = _B // _NW
_CH = 512
_NCHUNK = _BPW // _CH

_mesh = plsc.VectorSubcoreMesh(core_axis_name="c", subcore_axis_name="s")


@functools.partial(
    pl.kernel,
    mesh=_mesh,
    out_type=jax.ShapeDtypeStruct((_B, _EMBED_DIM), jnp.float32),
    scratch_types=[
        pltpu.VMEM((_CH,), jnp.int32),
        pltpu.VMEM((_CH, _EMBED_DIM), jnp.float32),
        pltpu.SemaphoreType.DMA,
    ],
    compiler_params=pltpu.CompilerParams(use_tc_tiling_on_sc=False),
)
def _gather_kernel(idx_hbm, table_hbm, out_hbm, idx_v, rows_v, sem):
    wid = lax.axis_index("s") * _NC + lax.axis_index("c")
    base = wid * _BPW

    def body(i, _):
        off = base + i * _CH
        pltpu.sync_copy(idx_hbm.at[pl.ds(off, _CH)], idx_v)
        pltpu.async_copy(table_hbm.at[idx_v], rows_v, sem).wait()
        pltpu.sync_copy(rows_v, out_hbm.at[pl.ds(off, _CH)])
        return 0

    lax.fori_loop(0, _NCHUNK, body, 0)


def kernel(input_ids, table):
    idx = input_ids.reshape(-1).astype(jnp.int32)
    out = _gather_kernel(idx, table)
    return out.reshape(_BATCH, _HIST, _EMBED_DIM)


# speedup vs baseline: 1.8687x; 1.0405x over previous
import functools

import jax
import jax.numpy as jnp
from jax import lax
from jax.experimental import pallas as pl
from jax.experimental.pallas import tpu as pltpu
from jax.experimental.pallas import tpu_sc as plsc

_VOCAB = 1000000
_EMBED_DIM = 64
_BATCH = 16384
_HIST = 50
_B = _BATCH * _HIST

_info = plsc.get_sparse_core_info()
_NC = _info.num_cores
_NS = _info.num_subcores
_NW = _NC * _NS
_BPW = _B // _NW
_CH = 512
_NCHUNK = _BPW // _CH

_mesh = plsc.VectorSubcoreMesh(core_axis_name="c", subcore_axis_name="s")


@functools.partial(
    pl.kernel,
    mesh=_mesh,
    out_type=jax.ShapeDtypeStruct((_B, _EMBED_DIM), jnp.float32),
    scratch_types=[
        pltpu.VMEM((_NCHUNK, _CH), jnp.int32),
        pltpu.VMEM((_CH, _EMBED_DIM), jnp.float32),
        pltpu.VMEM((_CH, _EMBED_DIM), jnp.float32),
        pltpu.SemaphoreType.DMA,
        pltpu.SemaphoreType.DMA,
        pltpu.SemaphoreType.DMA,
        pltpu.SemaphoreType.DMA,
    ],
    compiler_params=pltpu.CompilerParams(use_tc_tiling_on_sc=False),
)
def _gather_kernel(idx_hbm, table_hbm, out_hbm, idx_v, rows0, rows1,
                   gsem0, gsem1, wsem0, wsem1):
    wid = lax.axis_index("s") * _NC + lax.axis_index("c")
    base = wid * _BPW

    pltpu.sync_copy(idx_hbm.at[pl.ds(wid * _NCHUNK, _NCHUNK)], idx_v)

    rows = (rows0, rows1)
    gsems = (gsem0, gsem1)
    wsems = (wsem0, wsem1)

    def g_start(c, b):
        pltpu.async_copy(table_hbm.at[idx_v.at[c]], rows[b], gsems[b])

    def g_wait(c, b):
        pltpu.make_async_copy(table_hbm.at[idx_v.at[c]], rows[b],
                              gsems[b]).wait()

    def w_start(c, b):
        pltpu.async_copy(rows[b], out_hbm.at[pl.ds(base + c * _CH, _CH)],
                         wsems[b])

    def w_wait(c, b):
        pltpu.make_async_copy(rows[b], out_hbm.at[pl.ds(base + c * _CH, _CH)],
                              wsems[b]).wait()

    g_start(0, 0)

    def body(g, _):
        c0 = 2 * g
        c1 = c0 + 1
        g_wait(c0, 0)

        @pl.when(c0 >= 1)
        def _():
            w_wait(c0 - 1, 1)

        g_start(c1, 1)
        w_start(c0, 0)
        g_wait(c1, 1)
        w_wait(c0, 0)

        @pl.when(c1 + 1 < _NCHUNK)
        def _():
            g_start(c1 + 1, 0)

        w_start(c1, 1)
        return 0

    lax.fori_loop(0, _NCHUNK // 2, body, 0)
    w_wait(_NCHUNK - 1, 1)


def kernel(input_ids, table):
    idx = input_ids.reshape(_B // _CH, _CH).astype(jnp.int32)
    out = _gather_kernel(idx, table)
    return out.reshape(_BATCH, _HIST, _EMBED_DIM)


# speedup vs baseline: 1.8733x; 1.0024x over previous
import functools

import jax
import jax.numpy as jnp
from jax import lax
from jax.experimental import pallas as pl
from jax.experimental.pallas import tpu as pltpu
from jax.experimental.pallas import tpu_sc as plsc

_VOCAB = 1000000
_EMBED_DIM = 64
_BATCH = 16384
_HIST = 50
_B = _BATCH * _HIST

_info = plsc.get_sparse_core_info()
_NC = _info.num_cores
_NS = _info.num_subcores
_NW = _NC * _NS
_BPW = _B // _NW
_CH = 256
_NCHUNK = _BPW // _CH
_NBUF = 4
assert _NCHUNK % _NBUF == 0

_mesh = plsc.VectorSubcoreMesh(core_axis_name="c", subcore_axis_name="s")


@functools.partial(
    pl.kernel,
    mesh=_mesh,
    out_type=jax.ShapeDtypeStruct((_B, _EMBED_DIM), jnp.float32),
    scratch_types=[
        pltpu.VMEM((_NCHUNK, _CH), jnp.int32),
        [pltpu.VMEM((_CH, _EMBED_DIM), jnp.float32)] * _NBUF,
        [pltpu.SemaphoreType.DMA] * _NBUF,
        [pltpu.SemaphoreType.DMA] * _NBUF,
    ],
    compiler_params=pltpu.CompilerParams(use_tc_tiling_on_sc=False),
)
def _gather_kernel(idx_hbm, table_hbm, out_hbm, idx_v, rows, gsems, wsems):
    wid = lax.axis_index("s") * _NC + lax.axis_index("c")
    base = wid * _BPW

    pltpu.sync_copy(idx_hbm.at[pl.ds(wid * _NCHUNK, _NCHUNK)], idx_v)

    def g_start(c, b):
        pltpu.async_copy(table_hbm.at[idx_v.at[c]], rows[b], gsems[b])

    def g_wait(c, b):
        pltpu.make_async_copy(table_hbm.at[idx_v.at[c]], rows[b],
                              gsems[b]).wait()

    def w_start(c, b):
        pltpu.async_copy(rows[b], out_hbm.at[pl.ds(base + c * _CH, _CH)],
                         wsems[b])

    def w_wait(c, b):
        pltpu.make_async_copy(rows[b], out_hbm.at[pl.ds(base + c * _CH, _CH)],
                              wsems[b]).wait()

    for b in range(_NBUF - 1):
        g_start(b, b)

    def body(g, _):
        for b in range(_NBUF):
            c = g * _NBUF + b
            g_wait(c, b)
            w_start(c, b)
            n = c + _NBUF - 1
            nb = (b - 1) % _NBUF

            @pl.when(n < _NCHUNK)
            def _(c=c, n=n, nb=nb):
                @pl.when(n >= _NBUF)
                def _():
                    w_wait(n - _NBUF, nb)
                g_start(n, nb)
        return 0

    lax.fori_loop(0, _NCHUNK // _NBUF, body, 0)
    for b in range(_NBUF):
        c = _NCHUNK - _NBUF + b
        w_wait(c, b)


def kernel(input_ids, table):
    idx = input_ids.reshape(_B // _CH, _CH).astype(jnp.int32)
    out = _gather_kernel(idx, table)
    return out.reshape(_BATCH, _HIST, _EMBED_DIM)
